# baseline (speedup 1.0000x reference)
import functools

import jax
import jax.numpy as jnp
from jax import lax
from jax.experimental import pallas as pl
from jax.experimental.pallas import tpu as pltpu
from jax.experimental.pallas import tpu_sc as plsc

_N = 10000
_NPAD = 10112
_TRASH = _N
_E = 320000
_ETOT = _E + _N
_NW = 32
_B = 40
_STEPS = 260
_EW = _STEPS * _B
_EPAD = _NW * _EW
_ROWS_PER_TILE = _NPAD // 16
_DLEN = 2 * _NPAD + 16
_NEG_SLOPE = 0.2

_GATHER_DNUMS = lax.GatherDimensionNumbers(
    offset_dims=(), collapsed_slice_dims=(0,), start_index_map=(0,))


def _lane_shuffle(x, idx):
  return lax.gather(x, idx[:, None], _GATHER_DNUMS, slice_sizes=(1,),
                    mode=lax.GatherScatterMode.PROMISE_IN_BOUNDS)


def _make_edge_pass(nheads):
  kpb = 8 // nheads

  mesh = plsc.VectorSubcoreMesh(core_axis_name="c", subcore_axis_name="s")

  @functools.partial(
      pl.kernel,
      out_type=(
          jax.ShapeDtypeStruct((2, _NPAD, 128), jnp.float32),
          jax.ShapeDtypeStruct((_NW, _DLEN), jnp.float32),
      ),
      mesh=mesh,
      scratch_types=[
          pltpu.VMEM((2, _B), jnp.int32),
          pltpu.VMEM((2, _B), jnp.int32),
          pltpu.VMEM((_B, 128), jnp.float32),
          pltpu.VMEM((_B, 128), jnp.float32),
          pltpu.VMEM((_B, 128), jnp.float32),
          pltpu.VMEM((_B, 128), jnp.float32),
          pltpu.VMEM((_B, 128), jnp.float32),
          pltpu.VMEM((128,), jnp.float32),
          pltpu.VMEM((_DLEN,), jnp.float32),
          pltpu.VMEM_SHARED((_NPAD, 128), jnp.float32),
          pltpu.SemaphoreType.DMA,
          pltpu.SemaphoreType.DMA,
          pltpu.SemaphoreType.DMA,
          pltpu.SemaphoreType.DMA,
      ],
  )
  def edge_pass(eb_hbm, xl_hbm, xr_hbm, att_hbm, zrow_hbm, zd_hbm,
                u_out, d_out,
                ib0, ib1, rl0, rl1, rr0, rr1,
                msg, att_v, d_loc, u_sh, sl0, sl1, sr0, sr1):
    ibuf = (ib0, ib1)
    rows_l = (rl0, rl1)
    rows_r = (rr0, rr1)
    sem_l = (sl0, sl1)
    sem_r = (sr0, sr1)

    cid = lax.axis_index("c")
    sid = lax.axis_index("s")
    wid = sid * 2 + cid
    tile_base = sid * _ROWS_PER_TILE

    pltpu.sync_copy(att_hbm, att_v)
    pltpu.sync_copy(zd_hbm, d_loc)
    pltpu.sync_copy(zrow_hbm, u_sh.at[pl.ds(tile_base, _ROWS_PER_TILE)])
    plsc.subcore_barrier()

    cbase = wid * _STEPS
    lane = lax.iota(jnp.int32, 16)

    def issue(j, p):
      pltpu.sync_copy(eb_hbm.at[cbase + j], ibuf[p])
      pltpu.async_copy(xl_hbm.at[ibuf[p].at[1]], rows_l[p], sem_l[p])
      pltpu.async_copy(xr_hbm.at[ibuf[p].at[0]], rows_r[p], sem_r[p])

    def wait_gathers(p):
      pltpu.make_async_copy(xl_hbm.at[ibuf[p].at[1]], rows_l[p],
                            sem_l[p]).wait()
      pltpu.make_async_copy(xr_hbm.at[ibuf[p].at[0]], rows_r[p],
                            sem_r[p]).wait()

    def compute(p):
      rl = rows_l[p]
      rr = rows_r[p]
      ib = ibuf[p]

      def tree(v):
        while len(v) > 1:
          v = [v[i] + v[i + 1] for i in range(0, len(v), 2)]
        return v[0]

      def edge(e, att_c):
        l = [rl[e, pl.ds(k * 16, 16)] for k in range(8)]
        prods = []
        for k in range(8):
          a = l[k] + rr[e, pl.ds(k * 16, 16)]
          a = jnp.maximum(a, _NEG_SLOPE * a)
          prods.append(a * att_c[k])
        zero16 = jnp.zeros((16,), jnp.int32)
        if nheads == 2:
          a0 = tree(prods[:4])
          a1 = tree(prods[4:])
          t0 = a0 + _lane_shuffle(a0, lane ^ 8)
          t1 = a1 + _lane_shuffle(a1, lane ^ 8)
          m = jnp.where(lane < 8, t0, t1)
          for shift in (4, 2, 1):
            m = m + _lane_shuffle(m, lane ^ shift)
          z = jnp.exp(m)
          z1 = _lane_shuffle(z, zero16 + 8)
          zsplat = [_lane_shuffle(z, zero16)] * 4 + [z1] * 4
          zh = [jnp.where(lane == 0, z, 0.0), jnp.where(lane == 0, z1, 0.0)]
        else:
          m = tree(prods)
          for shift in (8, 4, 2, 1):
            m = m + _lane_shuffle(m, lane ^ shift)
          z = jnp.exp(m)
          zsplat = [z] * 8
          zh = [jnp.where(lane == 0, z, 0.0)]
        n = ib[0, pl.ds(e, 16)][0]
        for h in range(nheads):
          plsc.addupdate(d_loc.at[pl.ds(n + h * _NPAD, 16)], zh[h])
        for k in range(8):
          msg[e, pl.ds(k * 16, 16)] = l[k] * zsplat[k]
        return att_c

      att_c = tuple(att_v[pl.ds(k * 16, 16)] for k in range(8))
      lax.fori_loop(0, _B, edge, att_c, unroll=8)

    issue(0, 0)

    def body(i2, carry):
      for p in (0, 1):
        i = i2 * 2 + p
        nxt = i + 1

        @pl.when(nxt < _STEPS)
        def _prefetch():
          issue(nxt, 1 - p)

        wait_gathers(p)
        compute(p)
        pltpu.sync_copy(msg, u_sh.at[ibuf[p].at[0]], add=True)
      return carry

    lax.fori_loop(0, _STEPS // 2, body, 0)

    pltpu.sync_copy(d_loc, d_out.at[wid])
    plsc.subcore_barrier()
    pltpu.sync_copy(u_sh.at[pl.ds(tile_base, _ROWS_PER_TILE)],
                    u_out.at[cid, pl.ds(tile_base, _ROWS_PER_TILE)])

  return edge_pass


_edge_pass_h2 = _make_edge_pass(2)
_edge_pass_h1 = _make_edge_pass(1)


def _mm2_body(x_ref, wl_ref, wr_ref, ol_ref, or_ref):
  xb = x_ref[...]
  ol_ref[...] = jnp.dot(xb, wl_ref[...], preferred_element_type=jnp.float32)
  or_ref[...] = jnp.dot(xb, wr_ref[...], preferred_element_type=jnp.float32)


def _mm2(x, w_l, w_r):
  return pl.pallas_call(
      _mm2_body,
      grid=(4,),
      in_specs=[
          pl.BlockSpec((_NPAD // 4, 128), lambda i: (i, 0)),
          pl.BlockSpec((128, 128), lambda i: (0, 0)),
          pl.BlockSpec((128, 128), lambda i: (0, 0)),
      ],
      out_specs=[
          pl.BlockSpec((_NPAD // 4, 128), lambda i: (i, 0)),
          pl.BlockSpec((_NPAD // 4, 128), lambda i: (i, 0)),
      ],
      out_shape=[
          jax.ShapeDtypeStruct((_NPAD, 128), jnp.float32),
          jax.ShapeDtypeStruct((_NPAD, 128), jnp.float32),
      ],
  )(x, w_l, w_r)


def _combine_denoms(dp_ref):
  dsum = jnp.sum(dp_ref[...], axis=0)
  return jnp.transpose(dsum)


def _norm_mm_body(u_ref, dp_ref, b_ref, wl_ref, wr_ref, ol_ref, or_ref):
  u = u_ref[0] + u_ref[1]
  d = _combine_denoms(dp_ref)
  d0 = jnp.broadcast_to(d[:, 0:1], (_NPAD, 64))
  d1 = jnp.broadcast_to(d[:, 1:2], (_NPAD, 64))
  denom = jnp.concatenate([d0, d1], axis=1)
  h = u / (denom + 1e-16) + b_ref[...][None, :]
  h = jnp.where(h > 0, h, jnp.exp(h) - 1.0)
  ol_ref[...] = jnp.dot(h, wl_ref[...], preferred_element_type=jnp.float32)
  or_ref[...] = jnp.dot(h, wr_ref[...], preferred_element_type=jnp.float32)


def _norm_mm(u, dp, b, w_l, w_r):
  return pl.pallas_call(
      _norm_mm_body,
      in_specs=[
          pl.BlockSpec((2, _NPAD, 128), lambda: (0, 0, 0)),
          pl.BlockSpec((_NW, 2, _NPAD), lambda: (0, 0, 0)),
          pl.BlockSpec((128,), lambda: (0,)),
          pl.BlockSpec((128, 128), lambda: (0, 0)),
          pl.BlockSpec((128, 128), lambda: (0, 0)),
      ],
      out_specs=[
          pl.BlockSpec((_NPAD, 128), lambda: (0, 0)),
          pl.BlockSpec((_NPAD, 128), lambda: (0, 0)),
      ],
      out_shape=[
          jax.ShapeDtypeStruct((_NPAD, 128), jnp.float32),
          jax.ShapeDtypeStruct((_NPAD, 128), jnp.float32),
      ],
  )(u, dp[:, :2 * _NPAD].reshape(_NW, 2, _NPAD), b, w_l, w_r)


def _final_body(u_ref, dp_ref, b_ref, o_ref):
  u = u_ref[0] + u_ref[1]
  d = _combine_denoms(dp_ref)
  o_ref[...] = u[:_N] / (d[:_N, 0:1] + 1e-16) + b_ref[...][None, :]


def _final(u, dp, b):
  return pl.pallas_call(
      _final_body,
      in_specs=[
          pl.BlockSpec((2, _NPAD, 128), lambda: (0, 0, 0)),
          pl.BlockSpec((_NW, 2, _NPAD), lambda: (0, 0, 0)),
          pl.BlockSpec((128,), lambda: (0,)),
      ],
      out_specs=pl.BlockSpec((_N, 128), lambda: (0, 0)),
      out_shape=jax.ShapeDtypeStruct((_N, 128), jnp.float32),
  )(u, dp[:, :2 * _NPAD].reshape(_NW, 2, _NPAD), b)


def kernel(x, edge_index, edge_type, W1_l, W1_r, att1, b1, W2_l, W2_r, att2, b2):
  del edge_type
  idt = edge_index.dtype
  loop = jnp.arange(_N, dtype=idt)
  pad = _EPAD - _ETOT
  src = jnp.concatenate([edge_index[0], loop, jnp.zeros((pad,), idt)])
  dst = jnp.concatenate([edge_index[1], loop, jnp.full((pad,), _TRASH, idt)])
  x_pad = jnp.zeros((_NPAD, 128), jnp.float32).at[:_N].set(x)
  zrow = jnp.zeros((_ROWS_PER_TILE, 128), jnp.float32)
  zd = jnp.zeros((_DLEN,), jnp.float32)

  eb = jnp.stack([dst.reshape(-1, _B), src.reshape(-1, _B)], axis=1)

  xl1, xr1 = _mm2(x_pad, W1_l, W1_r)
  u1, dp1 = _edge_pass_h2(eb, xl1, xr1, att1.reshape(-1), zrow, zd)
  xl2, xr2 = _norm_mm(u1, dp1, b1, W2_l, W2_r)
  u2, dp2 = _edge_pass_h1(eb, xl2, xr2, att2.reshape(-1), zrow, zd)
  return _final(u2, dp2, b2)


# speedup vs baseline: 1.0966x; 1.0966x over previous
import functools

import jax
import jax.numpy as jnp
from jax import lax
from jax.experimental import pallas as pl
from jax.experimental.pallas import tpu as pltpu
from jax.experimental.pallas import tpu_sc as plsc

_N = 10000
_NPAD = 10112
_TRASH = _N
_E = 320000
_ETOT = _E + _N
_NW = 32
_B = 56
_STEPS = 186
_EW = _STEPS * _B
_EPAD = _NW * _EW
_ROWS_PER_TILE = _NPAD // 16
_DLEN = 2 * _NPAD + 16
_NEG_SLOPE = 0.2

_GATHER_DNUMS = lax.GatherDimensionNumbers(
    offset_dims=(), collapsed_slice_dims=(0,), start_index_map=(0,))


def _lane_shuffle(x, idx):
  return lax.gather(x, idx[:, None], _GATHER_DNUMS, slice_sizes=(1,),
                    mode=lax.GatherScatterMode.PROMISE_IN_BOUNDS)


def _make_edge_pass(nheads):
  kpb = 8 // nheads

  mesh = plsc.VectorSubcoreMesh(core_axis_name="c", subcore_axis_name="s")

  @functools.partial(
      pl.kernel,
      out_type=(
          jax.ShapeDtypeStruct((2, _NPAD, 128), jnp.float32),
          jax.ShapeDtypeStruct((_NW, _DLEN), jnp.float32),
      ),
      mesh=mesh,
      scratch_types=[
          pltpu.VMEM((2, _B), jnp.int32),
          pltpu.VMEM((2, _B), jnp.int32),
          pltpu.VMEM((_B, 128), jnp.float32),
          pltpu.VMEM((_B, 128), jnp.float32),
          pltpu.VMEM((_B, 128), jnp.float32),
          pltpu.VMEM((_B, 128), jnp.float32),
          pltpu.VMEM((128,), jnp.float32),
          pltpu.VMEM((_DLEN,), jnp.float32),
          pltpu.VMEM_SHARED((_NPAD, 128), jnp.float32),
          pltpu.SemaphoreType.DMA,
          pltpu.SemaphoreType.DMA,
          pltpu.SemaphoreType.DMA,
          pltpu.SemaphoreType.DMA,
          pltpu.SemaphoreType.DMA,
          pltpu.SemaphoreType.DMA,
      ],
  )
  def edge_pass(eb_hbm, xl_hbm, xr_hbm, att_hbm, zrow_hbm, zd_hbm,
                u_out, d_out,
                ib0, ib1, rl0, rl1, rr0, rr1,
                att_v, d_loc, u_sh, sl0, sl1, sr0, sr1, si0, si1):
    ibuf = (ib0, ib1)
    rows_l = (rl0, rl1)
    rows_r = (rr0, rr1)
    sem_l = (sl0, sl1)
    sem_r = (sr0, sr1)
    sem_i = (si0, si1)

    cid = lax.axis_index("c")
    sid = lax.axis_index("s")
    wid = sid * 2 + cid
    tile_base = sid * _ROWS_PER_TILE

    pltpu.sync_copy(att_hbm, att_v)
    pltpu.sync_copy(zd_hbm, d_loc)
    pltpu.sync_copy(zrow_hbm, u_sh.at[pl.ds(tile_base, _ROWS_PER_TILE)])
    plsc.subcore_barrier()

    cbase = wid * _STEPS
    lane = lax.iota(jnp.int32, 16)

    def idx_load(j, p):
      pltpu.async_copy(eb_hbm.at[cbase + j], ibuf[p], sem_i[p])

    def idx_wait(p):
      pltpu.make_async_copy(eb_hbm.at[cbase], ibuf[p], sem_i[p]).wait()

    def gathers(p):
      pltpu.async_copy(xl_hbm.at[ibuf[p].at[1]], rows_l[p], sem_l[p])
      pltpu.async_copy(xr_hbm.at[ibuf[p].at[0]], rows_r[p], sem_r[p])

    def wait_gathers(p):
      pltpu.make_async_copy(xl_hbm.at[ibuf[p].at[1]], rows_l[p],
                            sem_l[p]).wait()
      pltpu.make_async_copy(xr_hbm.at[ibuf[p].at[0]], rows_r[p],
                            sem_r[p]).wait()

    def compute(p):
      rl = rows_l[p]
      rr = rows_r[p]
      ib = ibuf[p]

      def tree(v):
        while len(v) > 1:
          v = [v[i] + v[i + 1] for i in range(0, len(v), 2)]
        return v[0]

      def edge(e, att_c):
        l = [rl[e, pl.ds(k * 16, 16)] for k in range(8)]
        prods = []
        for k in range(8):
          a = l[k] + rr[e, pl.ds(k * 16, 16)]
          a = jnp.maximum(a, _NEG_SLOPE * a)
          prods.append(a * att_c[k])
        zero16 = jnp.zeros((16,), jnp.int32)
        if nheads == 2:
          a0 = tree(prods[:4])
          a1 = tree(prods[4:])
          t0 = a0 + _lane_shuffle(a0, lane ^ 8)
          t1 = a1 + _lane_shuffle(a1, lane ^ 8)
          m = jnp.where(lane < 8, t0, t1)
          for shift in (4, 2, 1):
            m = m + _lane_shuffle(m, lane ^ shift)
          z = jnp.exp(m)
          z1 = _lane_shuffle(z, zero16 + 8)
          zsplat = [_lane_shuffle(z, zero16)] * 4 + [z1] * 4
          zh = [jnp.where(lane == 0, z, 0.0), jnp.where(lane == 0, z1, 0.0)]
        else:
          m = tree(prods)
          for shift in (8, 4, 2, 1):
            m = m + _lane_shuffle(m, lane ^ shift)
          z = jnp.exp(m)
          zsplat = [z] * 8
          zh = [jnp.where(lane == 0, z, 0.0)]
        n = ib[0, pl.ds(e, 16)][0]
        for h in range(nheads):
          plsc.addupdate(d_loc.at[pl.ds(n + h * _NPAD, 16)], zh[h])
        for k in range(8):
          rl[e, pl.ds(k * 16, 16)] = l[k] * zsplat[k]
        return att_c

      att_c = tuple(att_v[pl.ds(k * 16, 16)] for k in range(8))
      lax.fori_loop(0, _B, edge, att_c, unroll=8)

    idx_load(0, 0)
    idx_wait(0)
    gathers(0)
    idx_load(1, 1)

    def body(i2, carry):
      for p in (0, 1):
        i = i2 * 2 + p

        @pl.when(i + 1 < _STEPS)
        def _prefetch():
          idx_wait(1 - p)
          gathers(1 - p)

        wait_gathers(p)
        compute(p)
        pltpu.sync_copy(rows_l[p], u_sh.at[ibuf[p].at[0]], add=True)

        @pl.when(i + 2 < _STEPS)
        def _idx_prefetch():
          idx_load(i + 2, p)
      return carry

    lax.fori_loop(0, _STEPS // 2, body, 0)

    pltpu.sync_copy(d_loc, d_out.at[wid])
    plsc.subcore_barrier()
    pltpu.sync_copy(u_sh.at[pl.ds(tile_base, _ROWS_PER_TILE)],
                    u_out.at[cid, pl.ds(tile_base, _ROWS_PER_TILE)])

  return edge_pass


_edge_pass_h2 = _make_edge_pass(2)
_edge_pass_h1 = _make_edge_pass(1)


def _mm2_body(x_ref, wl_ref, wr_ref, ol_ref, or_ref):
  xb = x_ref[...]
  ol_ref[...] = jnp.dot(xb, wl_ref[...], preferred_element_type=jnp.float32)
  or_ref[...] = jnp.dot(xb, wr_ref[...], preferred_element_type=jnp.float32)


def _mm2(x, w_l, w_r):
  return pl.pallas_call(
      _mm2_body,
      grid=(4,),
      in_specs=[
          pl.BlockSpec((_NPAD // 4, 128), lambda i: (i, 0)),
          pl.BlockSpec((128, 128), lambda i: (0, 0)),
          pl.BlockSpec((128, 128), lambda i: (0, 0)),
      ],
      out_specs=[
          pl.BlockSpec((_NPAD // 4, 128), lambda i: (i, 0)),
          pl.BlockSpec((_NPAD // 4, 128), lambda i: (i, 0)),
      ],
      out_shape=[
          jax.ShapeDtypeStruct((_NPAD, 128), jnp.float32),
          jax.ShapeDtypeStruct((_NPAD, 128), jnp.float32),
      ],
  )(x, w_l, w_r)


def _combine_denoms(dp_ref):
  dsum = jnp.sum(dp_ref[...], axis=0)
  return jnp.transpose(dsum)


def _norm_mm_body(u_ref, dp_ref, b_ref, wl_ref, wr_ref, ol_ref, or_ref):
  u = u_ref[0] + u_ref[1]
  d = _combine_denoms(dp_ref)
  d0 = jnp.broadcast_to(d[:, 0:1], (_NPAD, 64))
  d1 = jnp.broadcast_to(d[:, 1:2], (_NPAD, 64))
  denom = jnp.concatenate([d0, d1], axis=1)
  h = u / (denom + 1e-16) + b_ref[...][None, :]
  h = jnp.where(h > 0, h, jnp.exp(h) - 1.0)
  ol_ref[...] = jnp.dot(h, wl_ref[...], preferred_element_type=jnp.float32)
  or_ref[...] = jnp.dot(h, wr_ref[...], preferred_element_type=jnp.float32)


def _norm_mm(u, dp, b, w_l, w_r):
  return pl.pallas_call(
      _norm_mm_body,
      in_specs=[
          pl.BlockSpec((2, _NPAD, 128), lambda: (0, 0, 0)),
          pl.BlockSpec((_NW, 2, _NPAD), lambda: (0, 0, 0)),
          pl.BlockSpec((128,), lambda: (0,)),
          pl.BlockSpec((128, 128), lambda: (0, 0)),
          pl.BlockSpec((128, 128), lambda: (0, 0)),
      ],
      out_specs=[
          pl.BlockSpec((_NPAD, 128), lambda: (0, 0)),
          pl.BlockSpec((_NPAD, 128), lambda: (0, 0)),
      ],
      out_shape=[
          jax.ShapeDtypeStruct((_NPAD, 128), jnp.float32),
          jax.ShapeDtypeStruct((_NPAD, 128), jnp.float32),
      ],
  )(u, dp[:, :2 * _NPAD].reshape(_NW, 2, _NPAD), b, w_l, w_r)


def _final_body(u_ref, dp_ref, b_ref, o_ref):
  u = u_ref[0] + u_ref[1]
  d = _combine_denoms(dp_ref)
  o_ref[...] = u[:_N] / (d[:_N, 0:1] + 1e-16) + b_ref[...][None, :]


def _final(u, dp, b):
  return pl.pallas_call(
      _final_body,
      in_specs=[
          pl.BlockSpec((2, _NPAD, 128), lambda: (0, 0, 0)),
          pl.BlockSpec((_NW, 2, _NPAD), lambda: (0, 0, 0)),
          pl.BlockSpec((128,), lambda: (0,)),
      ],
      out_specs=pl.BlockSpec((_N, 128), lambda: (0, 0)),
      out_shape=jax.ShapeDtypeStruct((_N, 128), jnp.float32),
  )(u, dp[:, :2 * _NPAD].reshape(_NW, 2, _NPAD), b)


def kernel(x, edge_index, edge_type, W1_l, W1_r, att1, b1, W2_l, W2_r, att2, b2):
  del edge_type
  idt = edge_index.dtype
  loop = jnp.arange(_N, dtype=idt)
  pad = _EPAD - _ETOT
  src = jnp.concatenate([edge_index[0], loop, jnp.zeros((pad,), idt)])
  dst = jnp.concatenate([edge_index[1], loop, jnp.full((pad,), _TRASH, idt)])
  x_pad = jnp.zeros((_NPAD, 128), jnp.float32).at[:_N].set(x)
  zrow = jnp.zeros((_ROWS_PER_TILE, 128), jnp.float32)
  zd = jnp.zeros((_DLEN,), jnp.float32)

  eb = jnp.stack([dst.reshape(-1, _B), src.reshape(-1, _B)], axis=1)

  xl1, xr1 = _mm2(x_pad, W1_l, W1_r)
  u1, dp1 = _edge_pass_h2(eb, xl1, xr1, att1.reshape(-1), zrow, zd)
  xl2, xr2 = _norm_mm(u1, dp1, b1, W2_l, W2_r)
  u2, dp2 = _edge_pass_h1(eb, xl2, xr2, att2.reshape(-1), zrow, zd)
  return _final(u2, dp2, b2)
